# baseline (speedup 1.0000x reference)
import jax
import jax.numpy as jnp
from jax import lax
from jax.experimental import pallas as pl
from jax.experimental.pallas import tpu as pltpu
from jax.experimental.pallas import tpu_sc as plsc

N_NODES = 10000
N_EDGES = 320000
D_FEAT = 128
D_EDGE = 16
OUT_DIM = 64
HALF = N_EDGES // 2

NC = 2
NS = 16
NW = NC * NS
LANES = 16
K = 80
EPW = N_EDGES // NW
NBLK = EPW // K
ROW = OUT_DIM + 16
RPT = 624


def _tc_node_body(nodes_ref, ws_ref, bs_ref, wr_ref, br_ref, wm_ref, bm_ref,
                  wa_ref, ts_ref, tr_ref, as_ref, ar_ref):
    n = nodes_ref[...]
    ts = jnp.dot(n, ws_ref[...], preferred_element_type=jnp.float32) + bs_ref[...]
    tr = jnp.dot(n, wr_ref[...], preferred_element_type=jnp.float32) + br_ref[...]
    tm = jnp.dot(n, wm_ref[...], preferred_element_type=jnp.float32) + bm_ref[...]
    ts_ref[...] = jnp.concatenate([ts, tm], axis=1)
    tr_ref[...] = tr
    wa = wa_ref[...]
    as_ref[...] = jnp.sum(ts * wa, axis=1).reshape(1, 1, -1)
    ar_ref[...] = jnp.sum(tr * wa, axis=1).reshape(1, 1, -1)


def _tc_node(nodes, W_sent1, b_sent1, W_recv, b_recv, W_msg, b_msg, W_attn):
    nb = 1000
    grid = (N_NODES // nb,)
    full = lambda shape: pl.BlockSpec(shape, lambda i: (0, 0))
    return pl.pallas_call(
        _tc_node_body,
        grid=grid,
        in_specs=[
            pl.BlockSpec((nb, D_FEAT), lambda i: (i, 0)),
            full((D_FEAT, OUT_DIM)), full((1, OUT_DIM)),
            full((D_FEAT, OUT_DIM)), full((1, OUT_DIM)),
            full((D_FEAT, OUT_DIM)), full((1, OUT_DIM)),
            full((1, OUT_DIM)),
        ],
        out_specs=[
            pl.BlockSpec((nb, 2 * OUT_DIM), lambda i: (i, 0)),
            pl.BlockSpec((nb, OUT_DIM), lambda i: (i, 0)),
            pl.BlockSpec((1, 1, nb), lambda i: (i, 0, 0)),
            pl.BlockSpec((1, 1, nb), lambda i: (i, 0, 0)),
        ],
        out_shape=[
            jax.ShapeDtypeStruct((N_NODES, 2 * OUT_DIM), jnp.float32),
            jax.ShapeDtypeStruct((N_NODES, OUT_DIM), jnp.float32),
            jax.ShapeDtypeStruct((N_NODES // nb, 1, nb), jnp.float32),
            jax.ShapeDtypeStruct((N_NODES // nb, 1, nb), jnp.float32),
        ],
    )(nodes, W_sent1, b_sent1.reshape(1, -1), W_recv, b_recv.reshape(1, -1),
      W_msg, b_msg.reshape(1, -1), W_attn.reshape(1, -1))


def _tc_edge_body(elo_ref, ehi_ref, we_ref, be_ref, wa_ref, ba_ref,
                  e2_ref, ae_ref):
    dn = (((0,), (0,)), ((), ()))
    lo = lax.dot_general(elo_ref[...], we_ref[...], dn,
                         preferred_element_type=jnp.float32) + be_ref[...]
    hi = lax.dot_general(ehi_ref[...], we_ref[...], dn,
                         preferred_element_type=jnp.float32) + be_ref[...]
    e2_ref[...] = jnp.concatenate([lo, hi], axis=1)
    wa = wa_ref[...]
    ba = ba_ref[...]
    aelo = jnp.sum(lo * wa, axis=1).reshape(1, -1) + ba
    aehi = jnp.sum(hi * wa, axis=1).reshape(1, -1) + ba
    ae_ref[...] = jnp.concatenate([aelo, aehi], axis=0).reshape(1, 2, -1)


def _tc_edge(edgesT, W_edge, b_edge, W_attn, b_attn):
    eb = 6400
    grid = (HALF // eb,)
    full = lambda shape: pl.BlockSpec(shape, lambda i: (0, 0))
    return pl.pallas_call(
        _tc_edge_body,
        grid=grid,
        in_specs=[
            pl.BlockSpec((D_EDGE, eb), lambda i: (0, i)),
            pl.BlockSpec((D_EDGE, eb), lambda i: (0, i + HALF // eb)),
            full((D_EDGE, OUT_DIM)), full((1, OUT_DIM)),
            full((1, OUT_DIM)), full((1, 1)),
        ],
        out_specs=[
            pl.BlockSpec((eb, 2 * OUT_DIM), lambda i: (i, 0)),
            pl.BlockSpec((1, 2, eb), lambda i: (i, 0, 0)),
        ],
        out_shape=[
            jax.ShapeDtypeStruct((HALF, 2 * OUT_DIM), jnp.float32),
            jax.ShapeDtypeStruct((HALF // eb, 2, eb), jnp.float32),
        ],
    )(edgesT, edgesT, W_edge, b_edge.reshape(1, -1), W_attn.reshape(1, -1),
      b_attn.reshape(1, 1))


def _tc_eft_body(ef2_ref, out_ref):
    h = pl.program_id(1)
    blk_t = jnp.transpose(ef2_ref[...])
    out_ref[...] = jnp.where(h == 0, blk_t[:OUT_DIM, :], blk_t[OUT_DIM:, :])


def _tc_eft(ef2):
    eb = 6400
    nblk = HALF // eb
    return pl.pallas_call(
        _tc_eft_body,
        grid=(nblk, 2),
        in_specs=[pl.BlockSpec((eb, 2 * OUT_DIM), lambda g, h: (g, 0))],
        out_specs=pl.BlockSpec((OUT_DIM, eb), lambda g, h: (0, h * nblk + g)),
        out_shape=jax.ShapeDtypeStruct((OUT_DIM, N_EDGES), jnp.float32),
    )(ef2)


def _tc_final_body(a0_ref, a1_ref, nodes_ref, wsf_ref, bsf_ref, out_ref):
    s = a0_ref[...] + a1_ref[...]
    vec = s[:, :OUT_DIM]
    den = s[:, OUT_DIM:OUT_DIM + 1]
    safe = jnp.where(den == 0.0, 1.0, den)
    self_part = jnp.dot(nodes_ref[...], wsf_ref[...],
                        preferred_element_type=jnp.float32) + bsf_ref[...]
    out_ref[...] = jnp.transpose(vec / safe + self_part)


def _tc_final(acc0, acc1, nodes, W_self, b_self):
    full = lambda shape: pl.BlockSpec(shape, lambda i: (0, 0))
    return pl.pallas_call(
        _tc_final_body,
        grid=(1,),
        in_specs=[
            full((N_NODES, ROW)),
            full((N_NODES, ROW)),
            full((N_NODES, D_FEAT)),
            full((D_FEAT, OUT_DIM)), full((1, OUT_DIM)),
        ],
        out_specs=full((OUT_DIM, N_NODES)),
        out_shape=jax.ShapeDtypeStruct((OUT_DIM, N_NODES), jnp.float32),
    )(acc0, acc1, nodes, W_self, b_self.reshape(1, -1))


def _sc_body(tsm_hbm, tr_hbm, e2_hbm, as_hbm, ar_hbm, ae_hbm,
             s2d_hbm, r2d_hbm,
             ef_out, acc_out,
             s2d_v, r2d_v, wbuf,
             abuf0, bbuf0, cbuf0, dbuf0, asb0, arb0, aeb0,
             abuf1, bbuf1, cbuf1, dbuf1, asb1, arb1, aeb1,
             isem0, isem1, osem0, osem1, acc_sh):
    c = lax.axis_index("c")
    sid = lax.axis_index("s")
    wid = sid * NC + c
    half_hi = wid >= NS
    hrow0 = (wid - jnp.where(half_hi, NS, 0)) * EPW
    sets = ((abuf0, bbuf0, cbuf0, dbuf0, asb0, arb0, aeb0, isem0, osem0),
            (abuf1, bbuf1, cbuf1, dbuf1, asb1, arb1, aeb1, isem1, osem1))
    dbuf = dbuf0

    pltpu.sync_copy(s2d_hbm.at[wid], s2d_v)
    pltpu.sync_copy(r2d_hbm.at[wid], r2d_v)

    zeros16 = jnp.zeros((LANES,), jnp.float32)

    def zero_row(k, _):
        for l in range(ROW // LANES):
            dbuf[k, pl.ds(l * LANES, LANES)] = zeros16
        return 0

    lax.fori_loop(0, K, zero_row, 0)
    base_row = sid * RPT
    nfull = RPT // K
    tail = RPT - nfull * K
    for i in range(nfull):
        pltpu.sync_copy(dbuf, acc_sh.at[pl.ds(base_row + i * K, K)])
    pltpu.sync_copy(dbuf.at[pl.ds(0, tail)],
                    acc_sh.at[pl.ds(base_row + nfull * K, tail)])

    extra = N_NODES - NS * RPT
    @pl.when(sid == NS - 1)
    def _():
        pltpu.sync_copy(dbuf.at[pl.ds(0, extra)],
                        acc_sh.at[pl.ds(NS * RPT, extra)])
    plsc.subcore_barrier()

    lane0 = lax.broadcasted_iota(jnp.int32, (LANES,), 0) == 0

    def start_in(blk, s):
        a, b, cb, _, asb, arb, aeb, isem, _ = sets[s]
        hrow = hrow0 + blk * K
        row = wid * NBLK + blk
        pltpu.async_copy(tsm_hbm.at[s2d_v.at[blk]], a, isem)
        pltpu.async_copy(tr_hbm.at[r2d_v.at[blk]], b, isem)
        pltpu.async_copy(as_hbm.at[s2d_v.at[blk]], asb, isem)
        pltpu.async_copy(ar_hbm.at[r2d_v.at[blk]], arb, isem)
        pltpu.async_copy(ae_hbm.at[pl.ds(row * K, K)], aeb, isem)

        @pl.when(jnp.logical_not(half_hi))
        def _():
            pltpu.async_copy(
                e2_hbm.at[pl.ds(hrow, K), pl.ds(0, OUT_DIM)], cb, isem)

        @pl.when(half_hi)
        def _():
            pltpu.async_copy(
                e2_hbm.at[pl.ds(hrow, K), pl.ds(OUT_DIM, OUT_DIM)], cb, isem)

    def phase(blk, s):
        a, b, cb, db, asb, arb, aeb, isem, osem = sets[s]
        _, _, cbo, dbo, _, _, _, _, osemo = sets[1 - s]
        hrow = hrow0 + blk * K

        @pl.when(blk > 0)
        def _():
            pltpu.make_async_copy(
                cbo, ef_out.at[pl.ds(0, K), pl.ds(0, OUT_DIM)], osemo).wait()
            pltpu.make_async_copy(dbo, acc_sh.at[pl.ds(0, K)], osemo).wait()

        @pl.when(blk + 1 < NBLK)
        def _():
            start_in(blk + 1, 1 - s)

        pltpu.make_async_copy(tsm_hbm.at[pl.ds(0, K)], a, isem).wait()
        for _ in range(2):
            pltpu.make_async_copy(e2_hbm.at[pl.ds(0, K), pl.ds(0, OUT_DIM)],
                                  cb, isem).wait()
        for _ in range(3):
            pltpu.make_async_copy(ae_hbm.at[pl.ds(0, K)], aeb, isem).wait()

        def ef_row(k, _):
            for l in range(OUT_DIM // LANES):
                sl = pl.ds(l * LANES, LANES)
                cb[k, sl] = a[k, sl] + b[k, sl] + cb[k, sl]
            return 0

        lax.fori_loop(0, K, ef_row, 0)

        @pl.when(jnp.logical_not(half_hi))
        def _():
            pltpu.async_copy(
                cb, ef_out.at[pl.ds(hrow, K), pl.ds(0, OUT_DIM)], osem)

        @pl.when(half_hi)
        def _():
            pltpu.async_copy(
                cb, ef_out.at[pl.ds(hrow, K), pl.ds(OUT_DIM, OUT_DIM)], osem)

        for j in range(K // LANES):
            sl = pl.ds(j * LANES, LANES)
            x = asb[sl] + arb[sl] + aeb[sl]
            x = jnp.where(x >= 0.0, x, 0.01 * x)
            wbuf[sl] = jnp.exp(x)

        def scale_group(j, _):
            wv16 = wbuf[pl.ds(j * LANES, LANES)]
            for k2 in range(LANES):
                k = j * LANES + k2
                wv = jnp.full((LANES,), wv16[k2], jnp.float32)
                for l in range(OUT_DIM // LANES):
                    db[k, pl.ds(l * LANES, LANES)] = (
                        a[k, pl.ds(OUT_DIM + l * LANES, LANES)] * wv)
                db[k, pl.ds(OUT_DIM, LANES)] = jnp.where(lane0, wv, 0.0)
            return 0

        lax.fori_loop(0, K // LANES, scale_group, 0)
        pltpu.async_copy(db, acc_sh.at[r2d_v.at[blk]], osem, add=True)

    start_in(0, 0)

    def pair(g, _):
        phase(2 * g, 0)
        phase(2 * g + 1, 1)
        return 0

    lax.fori_loop(0, NBLK // 2, pair, 0)
    phase(jnp.int32(NBLK - 1), 0)
    pltpu.make_async_copy(
        cbuf0, ef_out.at[pl.ds(0, K), pl.ds(0, OUT_DIM)], osem0).wait()
    pltpu.make_async_copy(dbuf0, acc_sh.at[pl.ds(0, K)], osem0).wait()

    plsc.subcore_barrier()
    pltpu.sync_copy(acc_sh.at[pl.ds(base_row, RPT)],
                    acc_out.at[c, pl.ds(base_row, RPT)])
    @pl.when(sid == NS - 1)
    def _():
        pltpu.sync_copy(acc_sh.at[pl.ds(NS * RPT, extra)],
                        acc_out.at[c, pl.ds(NS * RPT, extra)])


def _sc_call(tsm, tr, e2, a_s, a_r, ae, s2d, r2d):
    mesh = plsc.VectorSubcoreMesh(core_axis_name="c", subcore_axis_name="s")
    buf_set = [
        pltpu.VMEM((K, 2 * OUT_DIM), jnp.float32),
        pltpu.VMEM((K, OUT_DIM), jnp.float32),
        pltpu.VMEM((K, OUT_DIM), jnp.float32),
        pltpu.VMEM((K, ROW), jnp.float32),
        pltpu.VMEM((K,), jnp.float32),
        pltpu.VMEM((K,), jnp.float32),
        pltpu.VMEM((K,), jnp.float32),
    ]
    return pl.kernel(
        _sc_body,
        mesh=mesh,
        compiler_params=pltpu.CompilerParams(needs_layout_passes=False,
                                             use_tc_tiling_on_sc=False),
        out_type=[
            jax.ShapeDtypeStruct((HALF, 2 * OUT_DIM), jnp.float32),
            jax.ShapeDtypeStruct((NC, N_NODES, ROW), jnp.float32),
        ],
        scratch_types=[
            pltpu.VMEM((NBLK, K), jnp.int32),
            pltpu.VMEM((NBLK, K), jnp.int32),
            pltpu.VMEM((K,), jnp.float32),
        ] + buf_set + buf_set + [
            pltpu.SemaphoreType.DMA,
            pltpu.SemaphoreType.DMA,
            pltpu.SemaphoreType.DMA,
            pltpu.SemaphoreType.DMA,
            pltpu.VMEM_SHARED((N_NODES, ROW), jnp.float32),
        ],
    )(tsm, tr, e2, a_s, a_r, ae, s2d, r2d)


@jax.jit
def kernel(nodes, edges, senders, receivers, W_sent1, b_sent1, W_recv, b_recv,
           W_edge, b_edge, W_attn, b_attn, W_msg, b_msg, W_self, b_self):
    tsm, tr, a_s, a_r = _tc_node(nodes, W_sent1, b_sent1, W_recv, b_recv,
                                 W_msg, b_msg, W_attn)
    e2, ae2 = _tc_edge(edges.T, W_edge, b_edge, W_attn, b_attn)
    s2d = senders.reshape(NW, NBLK, K)
    r2d = receivers.reshape(NW, NBLK, K)
    ae = jnp.concatenate([ae2[:, 0, :].reshape(-1), ae2[:, 1, :].reshape(-1)])
    ef2, acc = _sc_call(tsm, tr, e2, a_s.reshape(-1), a_r.reshape(-1),
                        ae, s2d, r2d)
    ef = _tc_eft(ef2).T
    nf = _tc_final(acc[0], acc[1], nodes, W_self, b_self).T
    return nf, ef


# speedup vs baseline: 1.2337x; 1.2337x over previous
import jax
import jax.numpy as jnp
from jax import lax
from jax.experimental import pallas as pl
from jax.experimental.pallas import tpu as pltpu
from jax.experimental.pallas import tpu_sc as plsc

N_NODES = 10000
N_EDGES = 320000
D_FEAT = 128
D_EDGE = 16
OUT_DIM = 64
HALF = N_EDGES // 2

NC = 2
NS = 16
NW = NC * NS
LANES = 16
K = 80
EPW = N_EDGES // NW
NBLK = EPW // K
ROW = OUT_DIM + 16
RPT = 624


def _tc_node_body(nodes_ref, ws_ref, bs_ref, wr_ref, br_ref, wm_ref, bm_ref,
                  wa_ref, ts_ref, tr_ref, tm_ref, as_ref, ar_ref):
    n = nodes_ref[...]
    ts = jnp.dot(n, ws_ref[...], preferred_element_type=jnp.float32) + bs_ref[...]
    tr = jnp.dot(n, wr_ref[...], preferred_element_type=jnp.float32) + br_ref[...]
    ts_ref[...] = ts
    tr_ref[...] = tr
    tm_ref[...] = jnp.dot(n, wm_ref[...],
                          preferred_element_type=jnp.float32) + bm_ref[...]
    wa = wa_ref[...]
    as_ref[...] = jnp.sum(ts * wa, axis=1).reshape(1, 1, -1)
    ar_ref[...] = jnp.sum(tr * wa, axis=1).reshape(1, 1, -1)


def _tc_node(nodes, W_sent1, b_sent1, W_recv, b_recv, W_msg, b_msg, W_attn):
    nb = 1000
    grid = (N_NODES // nb,)
    full = lambda shape: pl.BlockSpec(shape, lambda i: (0, 0))
    return pl.pallas_call(
        _tc_node_body,
        grid=grid,
        in_specs=[
            pl.BlockSpec((nb, D_FEAT), lambda i: (i, 0)),
            full((D_FEAT, OUT_DIM)), full((1, OUT_DIM)),
            full((D_FEAT, OUT_DIM)), full((1, OUT_DIM)),
            full((D_FEAT, OUT_DIM)), full((1, OUT_DIM)),
            full((1, OUT_DIM)),
        ],
        out_specs=[
            pl.BlockSpec((nb, OUT_DIM), lambda i: (i, 0)),
            pl.BlockSpec((nb, OUT_DIM), lambda i: (i, 0)),
            pl.BlockSpec((nb, OUT_DIM), lambda i: (i, 0)),
            pl.BlockSpec((1, 1, nb), lambda i: (i, 0, 0)),
            pl.BlockSpec((1, 1, nb), lambda i: (i, 0, 0)),
        ],
        out_shape=[
            jax.ShapeDtypeStruct((N_NODES, OUT_DIM), jnp.float32),
            jax.ShapeDtypeStruct((N_NODES, OUT_DIM), jnp.float32),
            jax.ShapeDtypeStruct((N_NODES, OUT_DIM), jnp.float32),
            jax.ShapeDtypeStruct((N_NODES // nb, 1, nb), jnp.float32),
            jax.ShapeDtypeStruct((N_NODES // nb, 1, nb), jnp.float32),
        ],
    )(nodes, W_sent1, b_sent1.reshape(1, -1), W_recv, b_recv.reshape(1, -1),
      W_msg, b_msg.reshape(1, -1), W_attn.reshape(1, -1))


def _tc_edge_body(elo_ref, ehi_ref, we_ref, be_ref, wa_ref, ba_ref,
                  e2_ref, ae_ref):
    dn = (((0,), (0,)), ((), ()))
    lo = lax.dot_general(elo_ref[...], we_ref[...], dn,
                         preferred_element_type=jnp.float32) + be_ref[...]
    hi = lax.dot_general(ehi_ref[...], we_ref[...], dn,
                         preferred_element_type=jnp.float32) + be_ref[...]
    e2_ref[...] = jnp.concatenate([lo, hi], axis=1)
    wa = wa_ref[...]
    ba = ba_ref[...]
    aelo = jnp.sum(lo * wa, axis=1).reshape(1, -1) + ba
    aehi = jnp.sum(hi * wa, axis=1).reshape(1, -1) + ba
    ae_ref[...] = jnp.concatenate([aelo, aehi], axis=0).reshape(1, 2, -1)


def _tc_edge(edgesT, W_edge, b_edge, W_attn, b_attn):
    eb = 6400
    grid = (HALF // eb,)
    full = lambda shape: pl.BlockSpec(shape, lambda i: (0, 0))
    return pl.pallas_call(
        _tc_edge_body,
        grid=grid,
        in_specs=[
            pl.BlockSpec((D_EDGE, eb), lambda i: (0, i)),
            pl.BlockSpec((D_EDGE, eb), lambda i: (0, i + HALF // eb)),
            full((D_EDGE, OUT_DIM)), full((1, OUT_DIM)),
            full((1, OUT_DIM)), full((1, 1)),
        ],
        out_specs=[
            pl.BlockSpec((eb, 2 * OUT_DIM), lambda i: (i, 0)),
            pl.BlockSpec((1, 2, eb), lambda i: (i, 0, 0)),
        ],
        out_shape=[
            jax.ShapeDtypeStruct((HALF, 2 * OUT_DIM), jnp.float32),
            jax.ShapeDtypeStruct((HALF // eb, 2, eb), jnp.float32),
        ],
    )(edgesT, edgesT, W_edge, b_edge.reshape(1, -1), W_attn.reshape(1, -1),
      b_attn.reshape(1, 1))


def _tc_eft_body(ef2_ref, out_ref):
    h = pl.program_id(1)
    blk_t = jnp.transpose(ef2_ref[...])
    out_ref[...] = jnp.where(h == 0, blk_t[:OUT_DIM, :], blk_t[OUT_DIM:, :])


def _tc_eft(ef2):
    eb = 6400
    nblk = HALF // eb
    return pl.pallas_call(
        _tc_eft_body,
        grid=(nblk, 2),
        in_specs=[pl.BlockSpec((eb, 2 * OUT_DIM), lambda g, h: (g, 0))],
        out_specs=pl.BlockSpec((OUT_DIM, eb), lambda g, h: (0, h * nblk + g)),
        out_shape=jax.ShapeDtypeStruct((OUT_DIM, N_EDGES), jnp.float32),
    )(ef2)


def _tc_final_body(a0_ref, a1_ref, nodes_ref, wsf_ref, bsf_ref, out_ref):
    s = a0_ref[...] + a1_ref[...]
    vec = s[:, :OUT_DIM]
    den = s[:, OUT_DIM:OUT_DIM + 1]
    safe = jnp.where(den == 0.0, 1.0, den)
    self_part = jnp.dot(nodes_ref[...], wsf_ref[...],
                        preferred_element_type=jnp.float32) + bsf_ref[...]
    out_ref[...] = jnp.transpose(vec / safe + self_part)


def _tc_final(acc0, acc1, nodes, W_self, b_self):
    full = lambda shape: pl.BlockSpec(shape, lambda i: (0, 0))
    return pl.pallas_call(
        _tc_final_body,
        grid=(1,),
        in_specs=[
            full((N_NODES, ROW)),
            full((N_NODES, ROW)),
            full((N_NODES, D_FEAT)),
            full((D_FEAT, OUT_DIM)), full((1, OUT_DIM)),
        ],
        out_specs=full((OUT_DIM, N_NODES)),
        out_shape=jax.ShapeDtypeStruct((OUT_DIM, N_NODES), jnp.float32),
    )(acc0, acc1, nodes, W_self, b_self.reshape(1, -1))


def _sc_body(ts_hbm, tr_hbm, tm_hbm, e2_hbm, as_hbm, ar_hbm, ae_hbm,
             s2d_hbm, r2d_hbm,
             ef_out, acc_out,
             s2d_v, r2d_v, wbuf,
             abuf0, bbuf0, cbuf0, mbuf0, dbuf0, asb0, arb0, aeb0,
             abuf1, bbuf1, cbuf1, mbuf1, dbuf1, asb1, arb1, aeb1,
             isem0, isem1, osem0, osem1, acc_sh):
    c = lax.axis_index("c")
    sid = lax.axis_index("s")
    wid = sid * NC + c
    half_hi = wid >= NS
    hrow0 = (wid - jnp.where(half_hi, NS, 0)) * EPW
    sets = ((abuf0, bbuf0, cbuf0, mbuf0, dbuf0, asb0, arb0, aeb0,
             isem0, osem0),
            (abuf1, bbuf1, cbuf1, mbuf1, dbuf1, asb1, arb1, aeb1,
             isem1, osem1))
    dbuf = dbuf0

    pltpu.sync_copy(s2d_hbm.at[wid], s2d_v)
    pltpu.sync_copy(r2d_hbm.at[wid], r2d_v)

    zeros16 = jnp.zeros((LANES,), jnp.float32)

    def zero_row(k, _):
        for l in range(ROW // LANES):
            dbuf[k, pl.ds(l * LANES, LANES)] = zeros16
        return 0

    lax.fori_loop(0, K, zero_row, 0)
    base_row = sid * RPT
    nfull = RPT // K
    tail = RPT - nfull * K
    for i in range(nfull):
        pltpu.sync_copy(dbuf, acc_sh.at[pl.ds(base_row + i * K, K)])
    pltpu.sync_copy(dbuf.at[pl.ds(0, tail)],
                    acc_sh.at[pl.ds(base_row + nfull * K, tail)])

    extra = N_NODES - NS * RPT
    @pl.when(sid == NS - 1)
    def _():
        pltpu.sync_copy(dbuf.at[pl.ds(0, extra)],
                        acc_sh.at[pl.ds(NS * RPT, extra)])
    plsc.subcore_barrier()

    lane0 = lax.broadcasted_iota(jnp.int32, (LANES,), 0) == 0

    def start_in(blk, s):
        a, b, cb, m, _, asb, arb, aeb, isem, _ = sets[s]
        hrow = hrow0 + blk * K
        row = wid * NBLK + blk
        pltpu.async_copy(ts_hbm.at[s2d_v.at[blk]], a, isem)
        pltpu.async_copy(tr_hbm.at[r2d_v.at[blk]], b, isem)
        pltpu.async_copy(tm_hbm.at[s2d_v.at[blk]], m, isem)
        pltpu.async_copy(as_hbm.at[s2d_v.at[blk]], asb, isem)
        pltpu.async_copy(ar_hbm.at[r2d_v.at[blk]], arb, isem)
        pltpu.async_copy(ae_hbm.at[pl.ds(row * K, K)], aeb, isem)

        @pl.when(jnp.logical_not(half_hi))
        def _():
            pltpu.async_copy(
                e2_hbm.at[pl.ds(hrow, K), pl.ds(0, OUT_DIM)], cb, isem)

        @pl.when(half_hi)
        def _():
            pltpu.async_copy(
                e2_hbm.at[pl.ds(hrow, K), pl.ds(OUT_DIM, OUT_DIM)], cb, isem)

    def phase(blk, s):
        a, b, cb, m, db, asb, arb, aeb, isem, osem = sets[s]
        _, _, cbo, _, dbo, _, _, _, _, osemo = sets[1 - s]
        hrow = hrow0 + blk * K

        @pl.when(blk > 0)
        def _():
            pltpu.make_async_copy(
                cbo, ef_out.at[pl.ds(0, K), pl.ds(0, OUT_DIM)], osemo).wait()
            pltpu.make_async_copy(dbo, acc_sh.at[pl.ds(0, K)], osemo).wait()

        @pl.when(blk + 1 < NBLK)
        def _():
            start_in(blk + 1, 1 - s)

        for _ in range(4):
            pltpu.make_async_copy(tm_hbm.at[pl.ds(0, K)], a, isem).wait()
        for _ in range(3):
            pltpu.make_async_copy(ae_hbm.at[pl.ds(0, K)], aeb, isem).wait()

        def ef_row(k, _):
            for l in range(OUT_DIM // LANES):
                sl = pl.ds(l * LANES, LANES)
                cb[k, sl] = a[k, sl] + b[k, sl] + cb[k, sl]
            return 0

        lax.fori_loop(0, K, ef_row, 0)

        @pl.when(jnp.logical_not(half_hi))
        def _():
            pltpu.async_copy(
                cb, ef_out.at[pl.ds(hrow, K), pl.ds(0, OUT_DIM)], osem)

        @pl.when(half_hi)
        def _():
            pltpu.async_copy(
                cb, ef_out.at[pl.ds(hrow, K), pl.ds(OUT_DIM, OUT_DIM)], osem)

        for j in range(K // LANES):
            sl = pl.ds(j * LANES, LANES)
            x = asb[sl] + arb[sl] + aeb[sl]
            x = jnp.where(x >= 0.0, x, 0.01 * x)
            wbuf[sl] = jnp.exp(x)

        def scale_group(j, _):
            wv16 = wbuf[pl.ds(j * LANES, LANES)]
            for k2 in range(LANES):
                k = j * LANES + k2
                wv = jnp.full((LANES,), wv16[k2], jnp.float32)
                for l in range(OUT_DIM // LANES):
                    sl = pl.ds(l * LANES, LANES)
                    db[k, sl] = m[k, sl] * wv
                db[k, pl.ds(OUT_DIM, LANES)] = jnp.where(lane0, wv, 0.0)
            return 0

        lax.fori_loop(0, K // LANES, scale_group, 0)
        pltpu.async_copy(db, acc_sh.at[r2d_v.at[blk]], osem, add=True)

    start_in(0, 0)

    def pair(g, _):
        phase(2 * g, 0)
        phase(2 * g + 1, 1)
        return 0

    lax.fori_loop(0, NBLK // 2, pair, 0)
    phase(jnp.int32(NBLK - 1), 0)
    pltpu.make_async_copy(
        cbuf0, ef_out.at[pl.ds(0, K), pl.ds(0, OUT_DIM)], osem0).wait()
    pltpu.make_async_copy(dbuf0, acc_sh.at[pl.ds(0, K)], osem0).wait()

    plsc.subcore_barrier()
    pltpu.sync_copy(acc_sh.at[pl.ds(base_row, RPT)],
                    acc_out.at[c, pl.ds(base_row, RPT)])
    @pl.when(sid == NS - 1)
    def _():
        pltpu.sync_copy(acc_sh.at[pl.ds(NS * RPT, extra)],
                        acc_out.at[c, pl.ds(NS * RPT, extra)])


def _sc_call(ts, tr, tm, e2, a_s, a_r, ae, s2d, r2d):
    mesh = plsc.VectorSubcoreMesh(core_axis_name="c", subcore_axis_name="s")
    buf_set = [
        pltpu.VMEM((K, OUT_DIM), jnp.float32),
        pltpu.VMEM((K, OUT_DIM), jnp.float32),
        pltpu.VMEM((K, OUT_DIM), jnp.float32),
        pltpu.VMEM((K, OUT_DIM), jnp.float32),
        pltpu.VMEM((K, ROW), jnp.float32),
        pltpu.VMEM((K,), jnp.float32),
        pltpu.VMEM((K,), jnp.float32),
        pltpu.VMEM((K,), jnp.float32),
    ]
    return pl.kernel(
        _sc_body,
        mesh=mesh,
        compiler_params=pltpu.CompilerParams(needs_layout_passes=False,
                                             use_tc_tiling_on_sc=False),
        out_type=[
            jax.ShapeDtypeStruct((HALF, 2 * OUT_DIM), jnp.float32),
            jax.ShapeDtypeStruct((NC, N_NODES, ROW), jnp.float32),
        ],
        scratch_types=[
            pltpu.VMEM((NBLK, K), jnp.int32),
            pltpu.VMEM((NBLK, K), jnp.int32),
            pltpu.VMEM((K,), jnp.float32),
        ] + buf_set + buf_set + [
            pltpu.SemaphoreType.DMA,
            pltpu.SemaphoreType.DMA,
            pltpu.SemaphoreType.DMA,
            pltpu.SemaphoreType.DMA,
            pltpu.VMEM_SHARED((N_NODES, ROW), jnp.float32),
        ],
    )(ts, tr, tm, e2, a_s, a_r, ae, s2d, r2d)


@jax.jit
def kernel(nodes, edges, senders, receivers, W_sent1, b_sent1, W_recv, b_recv,
           W_edge, b_edge, W_attn, b_attn, W_msg, b_msg, W_self, b_self):
    ts, tr, tm, a_s, a_r = _tc_node(nodes, W_sent1, b_sent1, W_recv, b_recv,
                                    W_msg, b_msg, W_attn)
    e2, ae2 = _tc_edge(edges.T, W_edge, b_edge, W_attn, b_attn)
    s2d = senders.reshape(NW, NBLK, K)
    r2d = receivers.reshape(NW, NBLK, K)
    ae = jnp.concatenate([ae2[:, 0, :].reshape(-1), ae2[:, 1, :].reshape(-1)])
    ef2, acc = _sc_call(ts, tr, tm, e2, a_s.reshape(-1), a_r.reshape(-1),
                        ae, s2d, r2d)
    ef = _tc_eft(ef2).T
    nf = _tc_final(acc[0], acc[1], nodes, W_self, b_self).T
    return nf, ef
